# baseline (speedup 1.0000x reference)
import functools

import jax
import jax.numpy as jnp
from jax import lax
from jax.experimental import pallas as pl
from jax.experimental.pallas import tpu as pltpu
from jax.experimental.pallas import tpu_sc as plsc

_N = 10000
_D = 128
_E = 320000
_NP = 10240
_CHUNK = 128
_NC = 2
_NS = 16
_NW = _NC * _NS
_CPW = 80
_NCH = _NW * _CPW
_EPAD = _NCH * _CHUNK
_RED = _NP // _NS

_f32 = jnp.float32


def _sc_mesh():
  return plsc.VectorSubcoreMesh(core_axis_name="c", subcore_axis_name="s")


_DW = 128
_INFLIGHT = 8


def _deg_body(dst_hbm, out_hbm, deg_sh, dst_t, ones_v, ssem):
  c = lax.axis_index("c")
  s = lax.axis_index("s")
  w = c * _NS + s
  zeros16 = jnp.zeros((16,), _f32)
  ones16 = jnp.ones((16,), _f32)

  def fill(val16):
    def fill_body(i, _):
      for k in range(_DW // 16):
        ones_v[i, pl.ds(k * 16, 16)] = val16
      return 0
    lax.fori_loop(0, _CHUNK, fill_body, 0)

  fill(zeros16)
  for t in range(5):
    pltpu.sync_copy(ones_v,
                    deg_sh.at[pl.ds(s * _RED + t * _CHUNK, _CHUNK)])
  fill(ones16)
  pltpu.sync_copy(dst_hbm.at[pl.ds(w * _CPW, _CPW)], dst_t)
  plsc.subcore_barrier()

  def body(i, _):
    @pl.when(i > 0)
    def _waits():
      for _k in range(_INFLIGHT):
        pltpu.make_async_copy(ones_v, deg_sh.at[dst_t.at[0]], ssem).wait()
    for k in range(_INFLIGHT):
      pltpu.async_copy(ones_v, deg_sh.at[dst_t.at[_INFLIGHT * i + k]],
                       ssem, add=True)
    return 0

  lax.fori_loop(0, _CPW // _INFLIGHT, body, 0)
  for _k in range(_INFLIGHT):
    pltpu.make_async_copy(ones_v, deg_sh.at[dst_t.at[0]], ssem).wait()
  plsc.subcore_barrier()
  pltpu.sync_copy(deg_sh.at[pl.ds(s * _RED, _RED)],
                  out_hbm.at[c, pl.ds(s * _RED, _RED)])


@jax.jit
def _deg_call(dst):
  return pl.kernel(
      _deg_body,
      out_type=jax.ShapeDtypeStruct((_NC, _NP, _DW), _f32),
      mesh=_sc_mesh(),
      scratch_types=[
          pltpu.VMEM_SHARED((_NP, _DW), _f32),
          pltpu.VMEM((_CPW, _CHUNK), jnp.int32),
          pltpu.VMEM((_CHUNK, _DW), _f32),
          pltpu.SemaphoreType.DMA,
      ],
  )(dst)


_CK = 32
_CQ = _CPW * _CHUNK // _CK
_NB = 8
_PF = 4


def _agg_body(hs_hbm, ei_hbm, out_hbm, acc, *scratch):
  idx = scratch[:_NB]
  rows = scratch[_NB:2 * _NB]
  gs = scratch[2 * _NB:3 * _NB]
  ss = scratch[3 * _NB:4 * _NB]
  r0 = rows[0]
  c = lax.axis_index("c")
  s = lax.axis_index("s")
  w = c * _NS + s
  base = w * _CQ
  zeros16 = jnp.zeros((16,), _f32)

  def zero_rows(i, _):
    for k in range(_D // 16):
      r0[i, pl.ds(k * 16, 16)] = zeros16
    return 0

  lax.fori_loop(0, _CK, zero_rows, 0)
  for t in range(_RED // _CK):
    pltpu.sync_copy(r0, acc.at[pl.ds(s * _RED + t * _CK, _CK)])
  plsc.subcore_barrier()

  def prep(i, k, j):
    @pl.when(i > 0)
    def _drain():
      pltpu.make_async_copy(rows[k], acc.at[idx[k].at[1]], ss[k]).wait()

    pltpu.sync_copy(ei_hbm.at[base + j], idx[k])
    return pltpu.async_copy(hs_hbm.at[idx[k].at[0]], rows[k], gs[k])

  def fire(k, d):
    d.wait()
    pltpu.async_copy(rows[k], acc.at[idx[k].at[1]], ss[k], add=True)

  def body(i, _):
    j0 = _NB * i
    d = [None] * _NB
    for k in range(_PF):
      d[k] = prep(i, k, j0 + k)
    for k in range(_NB):
      fire(k, d[k])
      if k + _PF < _NB:
        d[k + _PF] = prep(i, k + _PF, j0 + k + _PF)
    return 0

  lax.fori_loop(0, _CQ // _NB, body, 0)
  for k in range(_NB):
    pltpu.make_async_copy(rows[k], acc.at[idx[k].at[1]], ss[k]).wait()
  plsc.subcore_barrier()
  pltpu.sync_copy(acc.at[pl.ds(s * _RED, _RED)],
                  out_hbm.at[c, pl.ds(s * _RED, _RED)])


@jax.jit
def _agg_call(hs, ei):
  return pl.kernel(
      _agg_body,
      out_type=jax.ShapeDtypeStruct((_NC, _NP, _D), _f32),
      mesh=_sc_mesh(),
      scratch_types=(
          [pltpu.VMEM_SHARED((_NP, _D), _f32)]
          + [pltpu.VMEM((2, _CK), jnp.int32)] * _NB
          + [pltpu.VMEM((_CK, _D), _f32)] * _NB
          + [pltpu.SemaphoreType.DMA] * (2 * _NB)
      ),
  )(hs, ei)


_R = 1000
_GRID = _N // _R


def _row_spec():
  return pl.BlockSpec((_R, _D), lambda i: (i, 0))


def _full_spec(shape):
  return pl.BlockSpec(shape, lambda i: tuple(0 for _ in shape))


def _mm_body(x_ref, w_ref, u_ref):
  u_ref[...] = jnp.dot(x_ref[...], w_ref[...], preferred_element_type=_f32)


@jax.jit
def _mm_call(x, w):
  return pl.pallas_call(
      _mm_body,
      grid=(_GRID,),
      in_specs=[_row_spec(), _full_spec((_D, _D))],
      out_specs=_row_spec(),
      out_shape=jax.ShapeDtypeStruct((_N, _D), _f32),
  )(x, w)


def _k1_body(u_ref, dp0_ref, dp1_ref, hs_ref, dis_ref):
  deg = dp0_ref[0, :, 0:1] + dp1_ref[0, :, 0:1] + 1.0
  dis = lax.rsqrt(deg)
  dis_ref[...] = dis
  hs_ref[...] = u_ref[...] * dis


@jax.jit
def _k1_call(u, degp):
  return pl.pallas_call(
      _k1_body,
      grid=(_GRID,),
      in_specs=[_row_spec(),
                pl.BlockSpec((1, _R, _DW), lambda i: (0, i, 0)),
                pl.BlockSpec((1, _R, _DW), lambda i: (1, i, 0))],
      out_specs=[_row_spec(), pl.BlockSpec((_R, 1), lambda i: (i, 0))],
      out_shape=[jax.ShapeDtypeStruct((_N, _D), _f32),
                 jax.ShapeDtypeStruct((_N, 1), _f32)],
  )(u, degp, degp)


def _agg_spec(core):
  return pl.BlockSpec((1, _R, _D), lambda i, c=core: (c, i, 0))


def _k2_body(a0_ref, a1_ref, hs_ref, dis_ref, b_ref, hp_ref, wn_ref,
             hn_ref, hsn_ref):
  s = a0_ref[0] + a1_ref[0] + hs_ref[...]
  pre = s * dis_ref[...] + b_ref[...]
  hn = jnp.maximum(pre, 0.0) + hp_ref[...]
  hn_ref[...] = hn
  hsn_ref[...] = jnp.dot(hn, wn_ref[...],
                         preferred_element_type=_f32) * dis_ref[...]


@jax.jit
def _k2_call(a, hs, dis, b, hp, wn):
  return pl.pallas_call(
      _k2_body,
      grid=(_GRID,),
      in_specs=[_agg_spec(0), _agg_spec(1), _row_spec(),
                pl.BlockSpec((_R, 1), lambda i: (i, 0)),
                _full_spec((1, _D)), _row_spec(), _full_spec((_D, _D))],
      out_specs=[_row_spec(), _row_spec()],
      out_shape=[jax.ShapeDtypeStruct((_N, _D), _f32),
                 jax.ShapeDtypeStruct((_N, _D), _f32)],
  )(a, a, hs, dis, b, hp, wn)


def _k3_body(a0_ref, a1_ref, hs_ref, dis_ref, b_ref, hp_ref, hn_ref):
  s = a0_ref[0] + a1_ref[0] + hs_ref[...]
  pre = s * dis_ref[...] + b_ref[...]
  hn_ref[...] = jnp.maximum(pre, 0.0) + hp_ref[...]


@jax.jit
def _k3_call(a, hs, dis, b, hp):
  return pl.pallas_call(
      _k3_body,
      grid=(_GRID,),
      in_specs=[_agg_spec(0), _agg_spec(1), _row_spec(),
                pl.BlockSpec((_R, 1), lambda i: (i, 0)),
                _full_spec((1, _D)), _row_spec()],
      out_specs=_row_spec(),
      out_shape=jax.ShapeDtypeStruct((_N, _D), _f32),
  )(a, a, hs, dis, b, hp)


def kernel(x, edge_index, W1, b1, W2, b2, W3, b3):
  pad = _EPAD - _E
  r = jnp.arange(pad, dtype=edge_index.dtype)
  src = jnp.concatenate([edge_index[0], (r * 97) % _N])
  dst = jnp.concatenate([edge_index[1], _N + (r % (_NP - _N))])
  ei = jnp.stack([src.reshape(-1, _CK), dst.reshape(-1, _CK)],
                 axis=1)
  dst = dst.reshape(_NCH, _CHUNK)

  u = _mm_call(x, W1)
  degp = _deg_call(dst)
  hs, dis = _k1_call(u, degp)
  h = x
  for (b, wn) in ((b1, W2), (b2, W3)):
    a = _agg_call(hs, ei)
    h, hs = _k2_call(a, hs, dis, b.reshape(1, _D), h, wn)
  a = _agg_call(hs, ei)
  h = _k3_call(a, hs, dis, b3.reshape(1, _D), h)
  return h


# speedup vs baseline: 1.2548x; 1.2548x over previous
import functools

import jax
import jax.numpy as jnp
from jax import lax
from jax.experimental import pallas as pl
from jax.experimental.pallas import tpu as pltpu
from jax.experimental.pallas import tpu_sc as plsc

_N = 10000
_D = 128
_E = 320000
_NP = 10240
_CHUNK = 128
_NC = 2
_NS = 16
_NW = _NC * _NS
_CPW = 80
_NCH = _NW * _CPW
_EPAD = _NCH * _CHUNK
_RED = _NP // _NS

_f32 = jnp.float32


def _sc_mesh():
  return plsc.VectorSubcoreMesh(core_axis_name="c", subcore_axis_name="s")


_DW = 128
_INFLIGHT = 4


def _deg_body(dst_hbm, out_hbm, deg_sh, dst_t, ones_v, ssem):
  c = lax.axis_index("c")
  s = lax.axis_index("s")
  w = c * _NS + s
  zeros16 = jnp.zeros((16,), _f32)
  ones16 = jnp.ones((16,), _f32)

  def fill(val16):
    def fill_body(i, _):
      for k in range(_DW // 16):
        ones_v[i, pl.ds(k * 16, 16)] = val16
      return 0
    lax.fori_loop(0, _CHUNK, fill_body, 0)

  fill(zeros16)
  for t in range(5):
    pltpu.sync_copy(ones_v,
                    deg_sh.at[pl.ds(s * _RED + t * _CHUNK, _CHUNK)])
  fill(ones16)
  pltpu.sync_copy(dst_hbm.at[pl.ds(w * _CPW, _CPW)], dst_t)
  plsc.subcore_barrier()

  def body(i, _):
    @pl.when(i > 0)
    def _waits():
      for _k in range(_INFLIGHT):
        pltpu.make_async_copy(ones_v, deg_sh.at[dst_t.at[0]], ssem).wait()
    for k in range(_INFLIGHT):
      pltpu.async_copy(ones_v, deg_sh.at[dst_t.at[_INFLIGHT * i + k]],
                       ssem, add=True)
    return 0

  lax.fori_loop(0, _CPW // _INFLIGHT, body, 0)
  for _k in range(_INFLIGHT):
    pltpu.make_async_copy(ones_v, deg_sh.at[dst_t.at[0]], ssem).wait()
  plsc.subcore_barrier()
  pltpu.sync_copy(deg_sh.at[pl.ds(s * _RED, _RED)],
                  out_hbm.at[c, pl.ds(s * _RED, _RED)])


@jax.jit
def _deg_call(dst):
  return pl.kernel(
      _deg_body,
      out_type=jax.ShapeDtypeStruct((_NC, _NP, _DW), _f32),
      mesh=_sc_mesh(),
      scratch_types=[
          pltpu.VMEM_SHARED((_NP, _DW), _f32),
          pltpu.VMEM((_CPW, _CHUNK), jnp.int32),
          pltpu.VMEM((_CHUNK, _DW), _f32),
          pltpu.SemaphoreType.DMA,
      ],
  )(dst)


_CK = 64
_CQ = _CPW * _CHUNK // _CK
_NB = 4
_PF = 2


def _agg_body(hs_hbm, ei_hbm, out_hbm, acc, *scratch):
  idx = scratch[:_NB]
  rows = scratch[_NB:2 * _NB]
  gs = scratch[2 * _NB:3 * _NB]
  ss = scratch[3 * _NB:4 * _NB]
  r0 = rows[0]
  c = lax.axis_index("c")
  s = lax.axis_index("s")
  w = c * _NS + s
  base = w * _CQ
  zeros16 = jnp.zeros((16,), _f32)

  def zero_rows(i, _):
    for k in range(_D // 16):
      r0[i, pl.ds(k * 16, 16)] = zeros16
    return 0

  lax.fori_loop(0, _CK, zero_rows, 0)
  for t in range(_RED // _CK):
    pltpu.sync_copy(r0, acc.at[pl.ds(s * _RED + t * _CK, _CK)])
  plsc.subcore_barrier()

  def prep(i, k, j):
    @pl.when(i > 0)
    def _drain():
      pltpu.make_async_copy(rows[k], acc.at[idx[k].at[1]], ss[k]).wait()

    pltpu.sync_copy(ei_hbm.at[base + j], idx[k])
    return pltpu.async_copy(hs_hbm.at[idx[k].at[0]], rows[k], gs[k])

  def fire(k, d):
    d.wait()
    pltpu.async_copy(rows[k], acc.at[idx[k].at[1]], ss[k], add=True)

  def body(i, _):
    j0 = _NB * i
    d = [None] * _NB
    for k in range(_PF):
      d[k] = prep(i, k, j0 + k)
    for k in range(_NB):
      fire(k, d[k])
      if k + _PF < _NB:
        d[k + _PF] = prep(i, k + _PF, j0 + k + _PF)
    return 0

  lax.fori_loop(0, _CQ // _NB, body, 0)
  for k in range(_NB):
    pltpu.make_async_copy(rows[k], acc.at[idx[k].at[1]], ss[k]).wait()
  plsc.subcore_barrier()
  pltpu.sync_copy(acc.at[pl.ds(s * _RED, _RED)],
                  out_hbm.at[c, pl.ds(s * _RED, _RED)])


@jax.jit
def _agg_call(hs, ei):
  return pl.kernel(
      _agg_body,
      out_type=jax.ShapeDtypeStruct((_NC, _NP, _D), _f32),
      mesh=_sc_mesh(),
      scratch_types=(
          [pltpu.VMEM_SHARED((_NP, _D), _f32)]
          + [pltpu.VMEM((2, _CK), jnp.int32)] * _NB
          + [pltpu.VMEM((_CK, _D), _f32)] * _NB
          + [pltpu.SemaphoreType.DMA] * (2 * _NB)
      ),
  )(hs, ei)


_R = 1000
_GRID = _N // _R


def _row_spec():
  return pl.BlockSpec((_R, _D), lambda i: (i, 0))


def _full_spec(shape):
  return pl.BlockSpec(shape, lambda i: tuple(0 for _ in shape))


def _mm_body(x_ref, w_ref, u_ref):
  u_ref[...] = jnp.dot(x_ref[...], w_ref[...], preferred_element_type=_f32)


@jax.jit
def _mm_call(x, w):
  return pl.pallas_call(
      _mm_body,
      grid=(_GRID,),
      in_specs=[_row_spec(), _full_spec((_D, _D))],
      out_specs=_row_spec(),
      out_shape=jax.ShapeDtypeStruct((_N, _D), _f32),
  )(x, w)


def _k1_body(u_ref, dp0_ref, dp1_ref, hs_ref, dis_ref):
  deg = dp0_ref[0, :, 0:1] + dp1_ref[0, :, 0:1] + 1.0
  dis = lax.rsqrt(deg)
  dis_ref[...] = dis
  hs_ref[...] = u_ref[...] * dis


@jax.jit
def _k1_call(u, degp):
  return pl.pallas_call(
      _k1_body,
      grid=(_GRID,),
      in_specs=[_row_spec(),
                pl.BlockSpec((1, _R, _DW), lambda i: (0, i, 0)),
                pl.BlockSpec((1, _R, _DW), lambda i: (1, i, 0))],
      out_specs=[_row_spec(), pl.BlockSpec((_R, 1), lambda i: (i, 0))],
      out_shape=[jax.ShapeDtypeStruct((_N, _D), _f32),
                 jax.ShapeDtypeStruct((_N, 1), _f32)],
  )(u, degp, degp)


def _agg_spec(core):
  return pl.BlockSpec((1, _R, _D), lambda i, c=core: (c, i, 0))


def _k2_body(a0_ref, a1_ref, hs_ref, dis_ref, b_ref, hp_ref, wn_ref,
             hn_ref, hsn_ref):
  s = a0_ref[0] + a1_ref[0] + hs_ref[...]
  pre = s * dis_ref[...] + b_ref[...]
  hn = jnp.maximum(pre, 0.0) + hp_ref[...]
  hn_ref[...] = hn
  hsn_ref[...] = jnp.dot(hn, wn_ref[...],
                         preferred_element_type=_f32) * dis_ref[...]


@jax.jit
def _k2_call(a, hs, dis, b, hp, wn):
  return pl.pallas_call(
      _k2_body,
      grid=(_GRID,),
      in_specs=[_agg_spec(0), _agg_spec(1), _row_spec(),
                pl.BlockSpec((_R, 1), lambda i: (i, 0)),
                _full_spec((1, _D)), _row_spec(), _full_spec((_D, _D))],
      out_specs=[_row_spec(), _row_spec()],
      out_shape=[jax.ShapeDtypeStruct((_N, _D), _f32),
                 jax.ShapeDtypeStruct((_N, _D), _f32)],
  )(a, a, hs, dis, b, hp, wn)


def _k3_body(a0_ref, a1_ref, hs_ref, dis_ref, b_ref, hp_ref, hn_ref):
  s = a0_ref[0] + a1_ref[0] + hs_ref[...]
  pre = s * dis_ref[...] + b_ref[...]
  hn_ref[...] = jnp.maximum(pre, 0.0) + hp_ref[...]


@jax.jit
def _k3_call(a, hs, dis, b, hp):
  return pl.pallas_call(
      _k3_body,
      grid=(_GRID,),
      in_specs=[_agg_spec(0), _agg_spec(1), _row_spec(),
                pl.BlockSpec((_R, 1), lambda i: (i, 0)),
                _full_spec((1, _D)), _row_spec()],
      out_specs=_row_spec(),
      out_shape=jax.ShapeDtypeStruct((_N, _D), _f32),
  )(a, a, hs, dis, b, hp)


def kernel(x, edge_index, W1, b1, W2, b2, W3, b3):
  pad = _EPAD - _E
  r = jnp.arange(pad, dtype=edge_index.dtype)
  src = jnp.concatenate([edge_index[0], (r * 97) % _N])
  dst = jnp.concatenate([edge_index[1], _N + (r % (_NP - _N))])
  ei = jnp.stack([src.reshape(-1, _CK), dst.reshape(-1, _CK)],
                 axis=1)
  dst = dst.reshape(_NCH, _CHUNK)

  u = _mm_call(x, W1)
  degp = _deg_call(dst)
  hs, dis = _k1_call(u, degp)
  h = x
  for (b, wn) in ((b1, W2), (b2, W3)):
    a = _agg_call(hs, ei)
    h, hs = _k2_call(a, hs, dis, b.reshape(1, _D), h, wn)
  a = _agg_call(hs, ei)
  h = _k3_call(a, hs, dis, b3.reshape(1, _D), h)
  return h


# speedup vs baseline: 1.3944x; 1.1112x over previous
import functools

import jax
import jax.numpy as jnp
from jax import lax
from jax.experimental import pallas as pl
from jax.experimental.pallas import tpu as pltpu
from jax.experimental.pallas import tpu_sc as plsc

_N = 10000
_D = 128
_E = 320000
_NP = 10240
_CHUNK = 128
_NC = 2
_NS = 16
_NW = _NC * _NS
_CPW = 80
_NCH = _NW * _CPW
_EPAD = _NCH * _CHUNK
_RED = _NP // _NS

_f32 = jnp.float32


def _sc_mesh():
  return plsc.VectorSubcoreMesh(core_axis_name="c", subcore_axis_name="s")


_DW = 128
_INFLIGHT = 4


def _deg_body(dst_hbm, out_hbm, deg_sh, dst_t, ones_v, ssem):
  c = lax.axis_index("c")
  s = lax.axis_index("s")
  w = c * _NS + s
  zeros16 = jnp.zeros((16,), _f32)
  ones16 = jnp.ones((16,), _f32)

  def fill(val16):
    def fill_body(i, _):
      for k in range(_DW // 16):
        ones_v[i, pl.ds(k * 16, 16)] = val16
      return 0
    lax.fori_loop(0, _CHUNK, fill_body, 0)

  fill(zeros16)
  for t in range(5):
    pltpu.sync_copy(ones_v,
                    deg_sh.at[pl.ds(s * _RED + t * _CHUNK, _CHUNK)])
  fill(ones16)
  pltpu.sync_copy(dst_hbm.at[pl.ds(w * _CPW, _CPW)], dst_t)
  plsc.subcore_barrier()

  def body(i, _):
    @pl.when(i > 0)
    def _waits():
      for _k in range(_INFLIGHT):
        pltpu.make_async_copy(ones_v, deg_sh.at[dst_t.at[0]], ssem).wait()
    for k in range(_INFLIGHT):
      pltpu.async_copy(ones_v, deg_sh.at[dst_t.at[_INFLIGHT * i + k]],
                       ssem, add=True)
    return 0

  lax.fori_loop(0, _CPW // _INFLIGHT, body, 0)
  for _k in range(_INFLIGHT):
    pltpu.make_async_copy(ones_v, deg_sh.at[dst_t.at[0]], ssem).wait()
  plsc.subcore_barrier()
  pltpu.sync_copy(deg_sh.at[pl.ds(s * _RED, _RED)],
                  out_hbm.at[c, pl.ds(s * _RED, _RED)])


@jax.jit
def _deg_call(dst):
  return pl.kernel(
      _deg_body,
      out_type=jax.ShapeDtypeStruct((_NC, _NP, _DW), _f32),
      mesh=_sc_mesh(),
      scratch_types=[
          pltpu.VMEM_SHARED((_NP, _DW), _f32),
          pltpu.VMEM((_CPW, _CHUNK), jnp.int32),
          pltpu.VMEM((_CHUNK, _DW), _f32),
          pltpu.SemaphoreType.DMA,
      ],
  )(dst)


_CK = 80
_CQ = _CPW * _CHUNK // _CK
_NB = 4
_PF = 2


def _agg_body(hs_hbm, ei_hbm, out_hbm, acc, *scratch):
  idx = scratch[:_NB]
  rows = scratch[_NB:2 * _NB]
  gs = scratch[2 * _NB:3 * _NB]
  ss = scratch[3 * _NB:4 * _NB]
  r0 = rows[0]
  c = lax.axis_index("c")
  s = lax.axis_index("s")
  w = c * _NS + s
  base = w * _CQ
  zeros16 = jnp.zeros((16,), _f32)

  def zero_rows(i, _):
    for k in range(_D // 16):
      r0[i, pl.ds(k * 16, 16)] = zeros16
    return 0

  lax.fori_loop(0, _CK, zero_rows, 0)
  for t in range(_RED // _CK):
    pltpu.sync_copy(r0, acc.at[pl.ds(s * _RED + t * _CK, _CK)])
  plsc.subcore_barrier()

  def prep(i, k, j):
    @pl.when(i > 0)
    def _drain():
      pltpu.make_async_copy(rows[k], acc.at[idx[k].at[1]], ss[k]).wait()

    pltpu.sync_copy(ei_hbm.at[base + j], idx[k])
    return pltpu.async_copy(hs_hbm.at[idx[k].at[0]], rows[k], gs[k])

  def fire(k, d):
    d.wait()
    pltpu.async_copy(rows[k], acc.at[idx[k].at[1]], ss[k], add=True)

  def body(i, _):
    j0 = _NB * i
    d = [None] * _NB
    for k in range(_PF):
      d[k] = prep(i, k, j0 + k)
    for k in range(_NB):
      fire(k, d[k])
      if k + _PF < _NB:
        d[k + _PF] = prep(i, k + _PF, j0 + k + _PF)
    return 0

  lax.fori_loop(0, _CQ // _NB, body, 0)
  for k in range(_NB):
    pltpu.make_async_copy(rows[k], acc.at[idx[k].at[1]], ss[k]).wait()
  plsc.subcore_barrier()
  pltpu.sync_copy(acc.at[pl.ds(s * _RED, _RED)],
                  out_hbm.at[c, pl.ds(s * _RED, _RED)])


@jax.jit
def _agg_call(hs, ei):
  return pl.kernel(
      _agg_body,
      out_type=jax.ShapeDtypeStruct((_NC, _NP, _D), _f32),
      mesh=_sc_mesh(),
      scratch_types=(
          [pltpu.VMEM_SHARED((_NP, _D), _f32)]
          + [pltpu.VMEM((2, _CK), jnp.int32)] * _NB
          + [pltpu.VMEM((_CK, _D), _f32)] * _NB
          + [pltpu.SemaphoreType.DMA] * (2 * _NB)
      ),
  )(hs, ei)


_R = 1000
_GRID = _N // _R


def _row_spec():
  return pl.BlockSpec((_R, _D), lambda i: (i, 0))


def _full_spec(shape):
  return pl.BlockSpec(shape, lambda i: tuple(0 for _ in shape))


def _mm_body(x_ref, w_ref, u_ref):
  u_ref[...] = jnp.dot(x_ref[...], w_ref[...], preferred_element_type=_f32)


@jax.jit
def _mm_call(x, w):
  return pl.pallas_call(
      _mm_body,
      grid=(_GRID,),
      in_specs=[_row_spec(), _full_spec((_D, _D))],
      out_specs=_row_spec(),
      out_shape=jax.ShapeDtypeStruct((_N, _D), _f32),
  )(x, w)


def _k1_body(u_ref, dp0_ref, dp1_ref, hs_ref, dis_ref):
  deg = dp0_ref[0, :, 0:1] + dp1_ref[0, :, 0:1] + 1.0
  dis = lax.rsqrt(deg)
  dis_ref[...] = dis
  hs_ref[...] = u_ref[...] * dis


@jax.jit
def _k1_call(u, degp):
  return pl.pallas_call(
      _k1_body,
      grid=(_GRID,),
      in_specs=[_row_spec(),
                pl.BlockSpec((1, _R, _DW), lambda i: (0, i, 0)),
                pl.BlockSpec((1, _R, _DW), lambda i: (1, i, 0))],
      out_specs=[_row_spec(), pl.BlockSpec((_R, 1), lambda i: (i, 0))],
      out_shape=[jax.ShapeDtypeStruct((_N, _D), _f32),
                 jax.ShapeDtypeStruct((_N, 1), _f32)],
  )(u, degp, degp)


def _agg_spec(core):
  return pl.BlockSpec((1, _R, _D), lambda i, c=core: (c, i, 0))


def _k2_body(a0_ref, a1_ref, hs_ref, dis_ref, b_ref, hp_ref, wn_ref,
             hn_ref, hsn_ref):
  s = a0_ref[0] + a1_ref[0] + hs_ref[...]
  pre = s * dis_ref[...] + b_ref[...]
  hn = jnp.maximum(pre, 0.0) + hp_ref[...]
  hn_ref[...] = hn
  hsn_ref[...] = jnp.dot(hn, wn_ref[...],
                         preferred_element_type=_f32) * dis_ref[...]


@jax.jit
def _k2_call(a, hs, dis, b, hp, wn):
  return pl.pallas_call(
      _k2_body,
      grid=(_GRID,),
      in_specs=[_agg_spec(0), _agg_spec(1), _row_spec(),
                pl.BlockSpec((_R, 1), lambda i: (i, 0)),
                _full_spec((1, _D)), _row_spec(), _full_spec((_D, _D))],
      out_specs=[_row_spec(), _row_spec()],
      out_shape=[jax.ShapeDtypeStruct((_N, _D), _f32),
                 jax.ShapeDtypeStruct((_N, _D), _f32)],
  )(a, a, hs, dis, b, hp, wn)


def _k3_body(a0_ref, a1_ref, hs_ref, dis_ref, b_ref, hp_ref, hn_ref):
  s = a0_ref[0] + a1_ref[0] + hs_ref[...]
  pre = s * dis_ref[...] + b_ref[...]
  hn_ref[...] = jnp.maximum(pre, 0.0) + hp_ref[...]


@jax.jit
def _k3_call(a, hs, dis, b, hp):
  return pl.pallas_call(
      _k3_body,
      grid=(_GRID,),
      in_specs=[_agg_spec(0), _agg_spec(1), _row_spec(),
                pl.BlockSpec((_R, 1), lambda i: (i, 0)),
                _full_spec((1, _D)), _row_spec()],
      out_specs=_row_spec(),
      out_shape=jax.ShapeDtypeStruct((_N, _D), _f32),
  )(a, a, hs, dis, b, hp)


def kernel(x, edge_index, W1, b1, W2, b2, W3, b3):
  pad = _EPAD - _E
  r = jnp.arange(pad, dtype=edge_index.dtype)
  src = jnp.concatenate([edge_index[0], (r * 97) % _N])
  dst = jnp.concatenate([edge_index[1], _N + (r % (_NP - _N))])
  ei = jnp.stack([src.reshape(-1, _CK), dst.reshape(-1, _CK)],
                 axis=1)
  dst = dst.reshape(_NCH, _CHUNK)

  u = _mm_call(x, W1)
  degp = _deg_call(dst)
  hs, dis = _k1_call(u, degp)
  h = x
  for (b, wn) in ((b1, W2), (b2, W3)):
    a = _agg_call(hs, ei)
    h, hs = _k2_call(a, hs, dis, b.reshape(1, _D), h, wn)
  a = _agg_call(hs, ei)
  h = _k3_call(a, hs, dis, b3.reshape(1, _D), h)
  return h
